# Initial kernel scaffold; baseline (speedup 1.0000x reference)
import jax
import jax.numpy as jnp
from jax.experimental import pallas as pl


def kernel(data, merge, structure, W_lin, b_lin, W_tr, b_tr):
    raise NotImplementedError("write your pallas kernel here")


# speedup vs baseline: 5.1365x; 5.1365x over previous
import functools

import jax
import jax.numpy as jnp
from jax import lax
from jax.experimental import pallas as pl
from jax.experimental.pallas import tpu as pltpu
from jax.experimental.pallas import tpu_sc as plsc

NC = 2
NS = 16
CH = 128


def _sc_segment_sum(table, src_p, dst_p, zeros_np, n_pad, d_aug, chunks_per_tile):
    rows_per_tile = n_pad // NS
    mesh = plsc.VectorSubcoreMesh(core_axis_name="c", subcore_axis_name="s")

    @functools.partial(
        pl.kernel,
        mesh=mesh,
        compiler_params=pltpu.CompilerParams(use_tc_tiling_on_sc=False),
        out_type=jax.ShapeDtypeStruct((NC, n_pad, d_aug), jnp.float32),
        scratch_types=[
            pltpu.VMEM((chunks_per_tile, CH), jnp.int32),
            pltpu.VMEM((chunks_per_tile, CH), jnp.int32),
            pltpu.VMEM((CH, d_aug), jnp.float32),
            pltpu.VMEM_SHARED((n_pad, d_aug), jnp.float32),
            pltpu.SemaphoreType.DMA,
        ],
    )
    def sc_kernel(table_hbm, src_hbm, dst_hbm, zeros_hbm, out_hbm,
                  src_v, dst_v, rows_v, acc_sh, sem):
        cid = lax.axis_index("c")
        sid = lax.axis_index("s")
        wid = cid * NS + sid
        base = sid * rows_per_tile
        pltpu.sync_copy(zeros_hbm.at[pl.ds(base, rows_per_tile)],
                        acc_sh.at[pl.ds(base, rows_per_tile)])
        pltpu.sync_copy(src_hbm.at[wid], src_v)
        pltpu.sync_copy(dst_hbm.at[wid], dst_v)
        plsc.subcore_barrier()

        @pl.loop(0, chunks_per_tile)
        def _(j):
            pltpu.async_copy(table_hbm.at[src_v.at[j]], rows_v, sem).wait()
            pltpu.sync_copy(rows_v, acc_sh.at[dst_v.at[j]], add=True)

        plsc.subcore_barrier()
        pltpu.sync_copy(acc_sh.at[pl.ds(base, rows_per_tile)],
                        out_hbm.at[cid].at[pl.ds(base, rows_per_tile)])

    return sc_kernel(table, src_p, dst_p, zeros_np)


def _combine(data, merge, p0, p1, w1t, b1, w2t, b2, waug, n, d, d_aug):
    br = 1000
    grid = (n // br,)

    def body(data_b, merge_b, p0_b, p1_b, w1t_b, b1_b, w2t_b, b2_b, waug_b, o_b):
        agg = p0_b[...] + p1_b[...]
        deg = agg[:, d:d + 1]
        out = jnp.dot(data_b[...], w1t_b[...],
                      preferred_element_type=jnp.float32,
                      precision=lax.Precision.HIGHEST) + b1_b[...]
        skip = jnp.dot(merge_b[...], w2t_b[...],
                       preferred_element_type=jnp.float32,
                       precision=lax.Precision.HIGHEST) + b2_b[...]
        msg = jnp.dot(agg, waug_b[...],
                      preferred_element_type=jnp.float32,
                      precision=lax.Precision.HIGHEST)
        lap = (deg * out - msg) / jnp.maximum(deg, 1.0)
        o_b[...] = jnp.maximum(lap + skip, 0.0)

    full = lambda shape: pl.BlockSpec(shape, lambda i: tuple(0 for _ in shape))
    return pl.pallas_call(
        body,
        grid=grid,
        in_specs=[
            pl.BlockSpec((br, d), lambda i: (i, 0)),
            pl.BlockSpec((br, d), lambda i: (i, 0)),
            pl.BlockSpec((br, d_aug), lambda i: (i, 0)),
            pl.BlockSpec((br, d_aug), lambda i: (i, 0)),
            full((d, d)),
            full((1, d)),
            full((d, d)),
            full((1, d)),
            full((d_aug, d)),
        ],
        out_specs=pl.BlockSpec((br, d), lambda i: (i, 0)),
        out_shape=jax.ShapeDtypeStruct((n, d), jnp.float32),
    )(data, merge, p0, p1, w1t, b1, w2t, b2, waug)


def kernel(data, merge, structure, W_lin, b_lin, W_tr, b_tr):
    n, d = data.shape
    e = structure.shape[1]
    d_aug = ((d + 1 + 15) // 16) * 16
    n_pad = ((n + 1 + 127) // 128) * 128
    chunks_per_tile = -(-e // (NC * NS * CH))
    e_pad = NC * NS * chunks_per_tile * CH

    table = jnp.concatenate(
        [data, jnp.ones((n, 1), jnp.float32),
         jnp.zeros((n, d_aug - d - 1), jnp.float32)], axis=1)
    src = structure[0]
    dst = structure[1]
    pad = e_pad - e
    src_p = jnp.concatenate([src, jnp.zeros((pad,), jnp.int32)]).reshape(
        NC * NS, chunks_per_tile, CH)
    dst_p = jnp.concatenate([dst, jnp.full((pad,), n, jnp.int32)]).reshape(
        NC * NS, chunks_per_tile, CH)
    zeros_np = jnp.zeros((n_pad, d_aug), jnp.float32)

    partials = _sc_segment_sum(table, src_p, dst_p, zeros_np,
                               n_pad, d_aug, chunks_per_tile)
    p0 = partials[0, :n]
    p1 = partials[1, :n]

    w1t = W_lin.T
    w2t = W_tr.T
    waug = jnp.concatenate(
        [W_lin.T, b_lin[None, :], jnp.zeros((d_aug - d - 1, d), jnp.float32)],
        axis=0)
    return _combine(data, merge, p0, p1, w1t, b_lin[None, :], w2t,
                    b_tr[None, :], waug, n, d, d_aug)


# speedup vs baseline: 5.3310x; 1.0379x over previous
import functools

import jax
import jax.numpy as jnp
from jax import lax
from jax.experimental import pallas as pl
from jax.experimental.pallas import tpu as pltpu
from jax.experimental.pallas import tpu_sc as plsc

NC = 2
NS = 16
CH = 64


def _sc_segment_sum(table, src_p, dst_p, zeros_np, n_pad, d_aug, chunks_per_tile):
    rows_per_tile = n_pad // NS
    mesh = plsc.VectorSubcoreMesh(core_axis_name="c", subcore_axis_name="s")

    @functools.partial(
        pl.kernel,
        mesh=mesh,
        compiler_params=pltpu.CompilerParams(use_tc_tiling_on_sc=False),
        out_type=jax.ShapeDtypeStruct((NC, n_pad, d_aug), jnp.float32),
        scratch_types=[
            pltpu.VMEM((chunks_per_tile, CH), jnp.int32),
            pltpu.VMEM((chunks_per_tile, CH), jnp.int32),
            pltpu.VMEM((CH, d_aug), jnp.float32),
            pltpu.VMEM((CH, d_aug), jnp.float32),
            pltpu.VMEM_SHARED((n_pad, d_aug), jnp.float32),
            pltpu.SemaphoreType.DMA,
            pltpu.SemaphoreType.DMA,
        ],
    )
    def sc_kernel(table_hbm, src_hbm, dst_hbm, zeros_hbm, out_hbm,
                  src_v, dst_v, rows0_v, rows1_v, acc_sh, sem0, sem1):
        cid = lax.axis_index("c")
        sid = lax.axis_index("s")
        wid = cid * NS + sid
        base = sid * rows_per_tile
        pltpu.sync_copy(zeros_hbm.at[pl.ds(base, rows_per_tile)],
                        acc_sh.at[pl.ds(base, rows_per_tile)])
        pltpu.sync_copy(src_hbm.at[wid], src_v)
        pltpu.sync_copy(dst_hbm.at[wid], dst_v)
        plsc.subcore_barrier()

        pltpu.async_copy(table_hbm.at[src_v.at[0]], rows0_v, sem0)

        @pl.loop(0, chunks_per_tile // 2)
        def _(k):
            j0 = 2 * k
            j1 = j0 + 1
            pltpu.make_async_copy(table_hbm.at[src_v.at[j0]], rows0_v,
                                  sem0).wait()
            pltpu.async_copy(table_hbm.at[src_v.at[j1]], rows1_v, sem1)
            pltpu.sync_copy(rows0_v, acc_sh.at[dst_v.at[j0]], add=True)
            pltpu.make_async_copy(table_hbm.at[src_v.at[j1]], rows1_v,
                                  sem1).wait()

            @pl.when(j1 + 1 < chunks_per_tile)
            def _():
                pltpu.async_copy(table_hbm.at[src_v.at[j1 + 1]], rows0_v, sem0)

            pltpu.sync_copy(rows1_v, acc_sh.at[dst_v.at[j1]], add=True)

        plsc.subcore_barrier()
        pltpu.sync_copy(acc_sh.at[pl.ds(base, rows_per_tile)],
                        out_hbm.at[cid].at[pl.ds(base, rows_per_tile)])

    return sc_kernel(table, src_p, dst_p, zeros_np)


def _combine(data, merge, p0, p1, w1t, b1, w2t, b2, waug, n, d, d_aug):
    br = 1000
    grid = (n // br,)

    def body(data_b, merge_b, p0_b, p1_b, w1t_b, b1_b, w2t_b, b2_b, waug_b, o_b):
        agg = p0_b[...] + p1_b[...]
        deg = agg[:, d:d + 1]
        out = jnp.dot(data_b[...], w1t_b[...],
                      preferred_element_type=jnp.float32,
                      precision=lax.Precision.HIGHEST) + b1_b[...]
        skip = jnp.dot(merge_b[...], w2t_b[...],
                       preferred_element_type=jnp.float32,
                       precision=lax.Precision.HIGHEST) + b2_b[...]
        msg = jnp.dot(agg, waug_b[...],
                      preferred_element_type=jnp.float32,
                      precision=lax.Precision.HIGHEST)
        lap = (deg * out - msg) / jnp.maximum(deg, 1.0)
        o_b[...] = jnp.maximum(lap + skip, 0.0)

    full = lambda shape: pl.BlockSpec(shape, lambda i: tuple(0 for _ in shape))
    return pl.pallas_call(
        body,
        grid=grid,
        in_specs=[
            pl.BlockSpec((br, d), lambda i: (i, 0)),
            pl.BlockSpec((br, d), lambda i: (i, 0)),
            pl.BlockSpec((br, d_aug), lambda i: (i, 0)),
            pl.BlockSpec((br, d_aug), lambda i: (i, 0)),
            full((d, d)),
            full((1, d)),
            full((d, d)),
            full((1, d)),
            full((d_aug, d)),
        ],
        out_specs=pl.BlockSpec((br, d), lambda i: (i, 0)),
        out_shape=jax.ShapeDtypeStruct((n, d), jnp.float32),
    )(data, merge, p0, p1, w1t, b1, w2t, b2, waug)


def kernel(data, merge, structure, W_lin, b_lin, W_tr, b_tr):
    n, d = data.shape
    e = structure.shape[1]
    d_aug = ((d + 1 + 15) // 16) * 16
    n_pad = ((n + 1 + 127) // 128) * 128
    chunks_per_tile = -(-e // (NC * NS * CH))
    chunks_per_tile += chunks_per_tile % 2
    e_pad = NC * NS * chunks_per_tile * CH

    table = jnp.concatenate(
        [data, jnp.ones((n, 1), jnp.float32),
         jnp.zeros((n, d_aug - d - 1), jnp.float32)], axis=1)
    src = structure[0]
    dst = structure[1]
    pad = e_pad - e
    src_p = jnp.concatenate([src, jnp.zeros((pad,), jnp.int32)]).reshape(
        NC * NS, chunks_per_tile, CH)
    dst_p = jnp.concatenate([dst, jnp.full((pad,), n, jnp.int32)]).reshape(
        NC * NS, chunks_per_tile, CH)
    zeros_np = jnp.zeros((n_pad, d_aug), jnp.float32)

    partials = _sc_segment_sum(table, src_p, dst_p, zeros_np,
                               n_pad, d_aug, chunks_per_tile)
    p0 = partials[0, :n]
    p1 = partials[1, :n]

    w1t = W_lin.T
    w2t = W_tr.T
    waug = jnp.concatenate(
        [W_lin.T, b_lin[None, :], jnp.zeros((d_aug - d - 1, d), jnp.float32)],
        axis=0)
    return _combine(data, merge, p0, p1, w1t, b_lin[None, :], w2t,
                    b_tr[None, :], waug, n, d, d_aug)


# speedup vs baseline: 11.2037x; 2.1016x over previous
import functools

import jax
import jax.numpy as jnp
from jax import lax
from jax.experimental import pallas as pl
from jax.experimental.pallas import tpu as pltpu
from jax.experimental.pallas import tpu_sc as plsc

NC = 2
NS = 16
CH = 80
DW = 16


def _sc_segment_sum(data, structure, n_pad, d):
    n, e = data.shape[0], structure.shape[1]
    del n
    edges_per_tile = e // (NC * NS)
    chunks = edges_per_tile // CH
    pairs = (chunks - 1) // 2
    rows_per_tile = n_pad // NS
    mesh = plsc.VectorSubcoreMesh(core_axis_name="c", subcore_axis_name="s")

    @functools.partial(
        pl.kernel,
        mesh=mesh,
        compiler_params=pltpu.CompilerParams(use_tc_tiling_on_sc=False),
        out_type=[
            jax.ShapeDtypeStruct((NC, n_pad, d), jnp.float32),
            jax.ShapeDtypeStruct((NC, n_pad, DW), jnp.float32),
        ],
        scratch_types=[
            pltpu.VMEM((2, CH), jnp.int32),
            pltpu.VMEM((2, CH), jnp.int32),
            pltpu.VMEM((CH, d), jnp.float32),
            pltpu.VMEM((CH, d), jnp.float32),
            pltpu.VMEM((CH, DW), jnp.float32),
            pltpu.VMEM((CH, DW), jnp.float32),
            pltpu.VMEM_SHARED((n_pad, d), jnp.float32),
            pltpu.VMEM_SHARED((n_pad, DW), jnp.float32),
            pltpu.SemaphoreType.DMA,
            pltpu.SemaphoreType.DMA,
            pltpu.SemaphoreType.DMA,
        ],
    )
    def sc_kernel(data_hbm, struct_hbm, out_hbm, deg_hbm,
                  idx0, idx1, rows0, rows1, ones_v, zdeg_v,
                  out_sh, deg_sh, sem0, sem1, semi):
        cid = lax.axis_index("c")
        sid = lax.axis_index("s")
        wid = cid * NS + sid
        ebase = wid * edges_per_tile
        rbase = sid * rows_per_tile

        @pl.loop(0, CH)
        def _(i):
            ones_v.at[pl.ds(i, 1), :][...] = jnp.ones((1, DW), jnp.float32)
            zdeg_v.at[pl.ds(i, 1), :][...] = jnp.zeros((1, DW), jnp.float32)

            @pl.loop(0, d, step=DW)
            def _(c):
                rows0.at[pl.ds(i, 1), pl.ds(c, DW)][...] = (
                    jnp.zeros((1, DW), jnp.float32))

        @pl.loop(0, 7)
        def _(i):
            pltpu.sync_copy(rows0, out_sh.at[pl.ds(rbase + i * CH, CH)])
            pltpu.sync_copy(zdeg_v, deg_sh.at[pl.ds(rbase + i * CH, CH)])
        pltpu.sync_copy(rows0.at[pl.ds(0, 72)],
                        out_sh.at[pl.ds(rbase + 7 * CH, 72)])
        pltpu.sync_copy(zdeg_v.at[pl.ds(0, 72)],
                        deg_sh.at[pl.ds(rbase + 7 * CH, 72)])
        plsc.subcore_barrier()

        def fetch_idx(buf, j):
            pltpu.async_copy(struct_hbm.at[0, pl.ds(ebase + j * CH, CH)],
                             buf.at[0], semi)
            pltpu.async_copy(struct_hbm.at[1, pl.ds(ebase + j * CH, CH)],
                             buf.at[1], semi)

        def wait_idx(buf, j):
            pltpu.make_async_copy(struct_hbm.at[0, pl.ds(ebase + j * CH, CH)],
                                  buf.at[0], semi).wait()
            pltpu.make_async_copy(struct_hbm.at[1, pl.ds(ebase + j * CH, CH)],
                                  buf.at[1], semi).wait()

        def gather(buf, rows, sem):
            pltpu.async_copy(data_hbm.at[buf.at[0]], rows, sem)

        def wait_gather(buf, rows, sem):
            pltpu.make_async_copy(data_hbm.at[buf.at[0]], rows, sem).wait()

        def scatter(buf, rows):
            pltpu.sync_copy(rows, out_sh.at[buf.at[1]], add=True)
            pltpu.sync_copy(ones_v, deg_sh.at[buf.at[1]], add=True)

        fetch_idx(idx0, 0)
        wait_idx(idx0, 0)
        gather(idx0, rows0, sem0)
        fetch_idx(idx1, 1)

        @pl.loop(0, pairs)
        def _(k):
            j1 = 2 * k + 1
            j2 = j1 + 1
            wait_gather(idx0, rows0, sem0)
            wait_idx(idx1, j1)
            gather(idx1, rows1, sem1)
            scatter(idx0, rows0)
            fetch_idx(idx0, j2)
            wait_gather(idx1, rows1, sem1)
            wait_idx(idx0, j2)
            gather(idx0, rows0, sem0)
            scatter(idx1, rows1)

            @pl.when(j2 + 1 < chunks)
            def _():
                fetch_idx(idx1, j2 + 1)

        wait_gather(idx0, rows0, sem0)
        scatter(idx0, rows0)

        plsc.subcore_barrier()
        pltpu.sync_copy(out_sh.at[pl.ds(rbase, rows_per_tile)],
                        out_hbm.at[cid].at[pl.ds(rbase, rows_per_tile)])
        pltpu.sync_copy(deg_sh.at[pl.ds(rbase, rows_per_tile)],
                        deg_hbm.at[cid].at[pl.ds(rbase, rows_per_tile)])

    return sc_kernel(data, structure)


def _mm(data, merge, w1t, b1, w2t, b2, n, d):
    br = 1000

    def body(data_b, merge_b, w1t_b, b1_b, w2t_b, b2_b, o_b, s_b):
        o_b[...] = jnp.dot(data_b[...], w1t_b[...],
                           preferred_element_type=jnp.float32,
                           precision=lax.Precision.HIGHEST) + b1_b[...]
        s_b[...] = jnp.dot(merge_b[...], w2t_b[...],
                           preferred_element_type=jnp.float32,
                           precision=lax.Precision.HIGHEST) + b2_b[...]

    full = lambda shape: pl.BlockSpec(shape, lambda i: tuple(0 for _ in shape))
    return pl.pallas_call(
        body,
        grid=(n // br,),
        in_specs=[
            pl.BlockSpec((br, d), lambda i: (i, 0)),
            pl.BlockSpec((br, d), lambda i: (i, 0)),
            full((d, d)), full((1, d)), full((d, d)), full((1, d)),
        ],
        out_specs=[pl.BlockSpec((br, d), lambda i: (i, 0)),
                   pl.BlockSpec((br, d), lambda i: (i, 0))],
        out_shape=[jax.ShapeDtypeStruct((n, d), jnp.float32),
                   jax.ShapeDtypeStruct((n, d), jnp.float32)],
    )(data, merge, w1t, b1, w2t, b2)


def _combine(out, skip, partials, degs, w1t, b1, n, d, n_pad):
    br = 1000

    def body(out_b, skip_b, p0_b, p1_b, d0_b, d1_b, w1t_b, b1_b, o_b):
        agg = p0_b[0] + p1_b[0]
        deg = d0_b[0, :, :1] + d1_b[0, :, :1]
        msg = jnp.dot(agg, w1t_b[...],
                      preferred_element_type=jnp.float32,
                      precision=lax.Precision.HIGHEST) + deg * b1_b[...]
        lap = (deg * out_b[...] - msg) / jnp.maximum(deg, 1.0)
        o_b[...] = jnp.maximum(lap + skip_b[...], 0.0)

    full = lambda shape: pl.BlockSpec(shape, lambda i: tuple(0 for _ in shape))
    return pl.pallas_call(
        body,
        grid=(n // br,),
        in_specs=[
            pl.BlockSpec((br, d), lambda i: (i, 0)),
            pl.BlockSpec((br, d), lambda i: (i, 0)),
            pl.BlockSpec((1, br, d), lambda i: (0, i, 0)),
            pl.BlockSpec((1, br, d), lambda i: (1, i, 0)),
            pl.BlockSpec((1, br, DW), lambda i: (0, i, 0)),
            pl.BlockSpec((1, br, DW), lambda i: (1, i, 0)),
            full((d, d)), full((1, d)),
        ],
        out_specs=pl.BlockSpec((br, d), lambda i: (i, 0)),
        out_shape=jax.ShapeDtypeStruct((n, d), jnp.float32),
    )(out, skip, partials, partials, degs, degs, w1t, b1)


def kernel(data, merge, structure, W_lin, b_lin, W_tr, b_tr):
    n, d = data.shape
    n_pad = ((n + 127) // 128) * 128

    w1t = W_lin.T
    w2t = W_tr.T
    out, skip = _mm(data, merge, w1t, b_lin[None, :], w2t, b_tr[None, :], n, d)
    partials, degs = _sc_segment_sum(data, structure, n_pad, d)
    return _combine(out, skip, partials, degs,
                    w1t, b_lin[None, :], n, d, n_pad)


# speedup vs baseline: 11.8694x; 1.0594x over previous
import functools

import jax
import jax.numpy as jnp
from jax import lax
from jax.experimental import pallas as pl
from jax.experimental.pallas import tpu as pltpu
from jax.experimental.pallas import tpu_sc as plsc

NC = 2
NS = 16
CH = 128
TL = 16
DW = 16


def _sc_segment_sum(data, structure, n_pad, d):
    n, e = data.shape[0], structure.shape[1]
    del n
    edges_per_tile = e // (NC * NS)
    chunks = edges_per_tile // CH
    assert chunks * CH + TL == edges_per_tile and chunks % 2 == 0
    pairs = chunks // 2
    rows_per_tile = n_pad // NS
    mesh = plsc.VectorSubcoreMesh(core_axis_name="c", subcore_axis_name="s")

    @functools.partial(
        pl.kernel,
        mesh=mesh,
        compiler_params=pltpu.CompilerParams(use_tc_tiling_on_sc=False),
        out_type=[
            jax.ShapeDtypeStruct((NC, n_pad, d), jnp.float32),
            jax.ShapeDtypeStruct((NC, n_pad, DW), jnp.float32),
        ],
        scratch_types=[
            pltpu.VMEM((2, CH), jnp.int32),
            pltpu.VMEM((2, CH), jnp.int32),
            pltpu.VMEM((CH, d), jnp.float32),
            pltpu.VMEM((CH, d), jnp.float32),
            pltpu.VMEM((CH, DW), jnp.float32),
            pltpu.VMEM((CH, DW), jnp.float32),
            pltpu.VMEM((2, TL), jnp.int32),
            pltpu.VMEM((TL, d), jnp.float32),
            pltpu.VMEM_SHARED((n_pad, d), jnp.float32),
            pltpu.VMEM_SHARED((n_pad, DW), jnp.float32),
            pltpu.SemaphoreType.DMA,
            pltpu.SemaphoreType.DMA,
            pltpu.SemaphoreType.DMA,
            pltpu.SemaphoreType.DMA,
            pltpu.SemaphoreType.DMA,
        ],
    )
    def sc_kernel(data_hbm, struct_hbm, out_hbm, deg_hbm,
                  idx0, idx1, rows0, rows1, ones_v, zdeg_v, idxt, rowst,
                  out_sh, deg_sh, sem0, sem1, semi, ssem0, ssem1):
        cid = lax.axis_index("c")
        sid = lax.axis_index("s")
        wid = cid * NS + sid
        ebase = wid * edges_per_tile
        rbase = sid * rows_per_tile

        @pl.loop(0, CH)
        def _(i):
            ones_v.at[pl.ds(i, 1), :][...] = jnp.ones((1, DW), jnp.float32)
            zdeg_v.at[pl.ds(i, 1), :][...] = jnp.zeros((1, DW), jnp.float32)

            @pl.loop(0, d, step=DW)
            def _(c):
                rows0.at[pl.ds(i, 1), pl.ds(c, DW)][...] = (
                    jnp.zeros((1, DW), jnp.float32))

        @pl.loop(0, 4)
        def _(i):
            pltpu.sync_copy(rows0, out_sh.at[pl.ds(rbase + i * CH, CH)])
            pltpu.sync_copy(zdeg_v, deg_sh.at[pl.ds(rbase + i * CH, CH)])
        pltpu.sync_copy(rows0.at[pl.ds(0, 120)],
                        out_sh.at[pl.ds(rbase + 4 * CH, 120)])
        pltpu.sync_copy(zdeg_v.at[pl.ds(0, 120)],
                        deg_sh.at[pl.ds(rbase + 4 * CH, 120)])
        plsc.subcore_barrier()

        def fetch_idx(buf, j):
            pltpu.async_copy(struct_hbm.at[0, pl.ds(ebase + j * CH, CH)],
                             buf.at[0], semi)
            pltpu.async_copy(struct_hbm.at[1, pl.ds(ebase + j * CH, CH)],
                             buf.at[1], semi)

        def wait_idx(buf, j):
            pltpu.make_async_copy(struct_hbm.at[0, pl.ds(ebase + j * CH, CH)],
                                  buf.at[0], semi).wait()
            pltpu.make_async_copy(struct_hbm.at[1, pl.ds(ebase + j * CH, CH)],
                                  buf.at[1], semi).wait()

        def gather(buf, rows, sem):
            pltpu.async_copy(data_hbm.at[buf.at[0]], rows, sem)

        def wait_gather(buf, rows, sem):
            pltpu.make_async_copy(data_hbm.at[buf.at[0]], rows, sem).wait()

        def scatter(buf, rows, ssem):
            pltpu.async_copy(rows, out_sh.at[buf.at[1]], ssem, add=True)
            pltpu.async_copy(ones_v, deg_sh.at[buf.at[1]], ssem, add=True)

        def wait_scatter(buf, rows, ssem):
            pltpu.make_async_copy(rows, out_sh.at[buf.at[1]], ssem).wait()
            pltpu.make_async_copy(ones_v, deg_sh.at[buf.at[1]], ssem).wait()

        fetch_idx(idx0, 0)
        wait_idx(idx0, 0)
        gather(idx0, rows0, sem0)
        fetch_idx(idx1, 1)

        @pl.loop(0, pairs)
        def _(k):
            j0 = 2 * k
            j1 = j0 + 1
            j2 = j0 + 2

            @pl.when(k > 0)
            def _():
                wait_scatter(idx1, rows1, ssem1)
                fetch_idx(idx1, j1)

            wait_gather(idx0, rows0, sem0)
            scatter(idx0, rows0, ssem0)
            wait_idx(idx1, j1)
            gather(idx1, rows1, sem1)

            wait_gather(idx1, rows1, sem1)
            wait_scatter(idx0, rows0, ssem0)

            @pl.when(j2 < chunks)
            def _():
                fetch_idx(idx0, j2)

            scatter(idx1, rows1, ssem1)

            @pl.when(j2 < chunks)
            def _():
                wait_idx(idx0, j2)
                gather(idx0, rows0, sem0)

        pltpu.async_copy(
            struct_hbm.at[0, pl.ds(ebase + chunks * CH, TL)], idxt.at[0], semi)
        pltpu.async_copy(
            struct_hbm.at[1, pl.ds(ebase + chunks * CH, TL)], idxt.at[1], semi)
        pltpu.make_async_copy(
            struct_hbm.at[0, pl.ds(ebase + chunks * CH, TL)], idxt.at[0],
            semi).wait()
        pltpu.make_async_copy(
            struct_hbm.at[1, pl.ds(ebase + chunks * CH, TL)], idxt.at[1],
            semi).wait()
        pltpu.async_copy(data_hbm.at[idxt.at[0]], rowst, sem0).wait()
        pltpu.sync_copy(rowst, out_sh.at[idxt.at[1]], add=True)
        pltpu.sync_copy(ones_v.at[pl.ds(0, TL)], deg_sh.at[idxt.at[1]],
                        add=True)
        wait_scatter(idx1, rows1, ssem1)

        plsc.subcore_barrier()
        pltpu.sync_copy(out_sh.at[pl.ds(rbase, rows_per_tile)],
                        out_hbm.at[cid].at[pl.ds(rbase, rows_per_tile)])
        pltpu.sync_copy(deg_sh.at[pl.ds(rbase, rows_per_tile)],
                        deg_hbm.at[cid].at[pl.ds(rbase, rows_per_tile)])

    return sc_kernel(data, structure)


def _mm(data, merge, w1t, b1, w2t, b2, n, d):
    br = 1000

    def body(data_b, merge_b, w1t_b, b1_b, w2t_b, b2_b, o_b, s_b):
        o_b[...] = jnp.dot(data_b[...], w1t_b[...],
                           preferred_element_type=jnp.float32,
                           precision=lax.Precision.HIGHEST) + b1_b[...]
        s_b[...] = jnp.dot(merge_b[...], w2t_b[...],
                           preferred_element_type=jnp.float32,
                           precision=lax.Precision.HIGHEST) + b2_b[...]

    full = lambda shape: pl.BlockSpec(shape, lambda i: tuple(0 for _ in shape))
    return pl.pallas_call(
        body,
        grid=(n // br,),
        in_specs=[
            pl.BlockSpec((br, d), lambda i: (i, 0)),
            pl.BlockSpec((br, d), lambda i: (i, 0)),
            full((d, d)), full((1, d)), full((d, d)), full((1, d)),
        ],
        out_specs=[pl.BlockSpec((br, d), lambda i: (i, 0)),
                   pl.BlockSpec((br, d), lambda i: (i, 0))],
        out_shape=[jax.ShapeDtypeStruct((n, d), jnp.float32),
                   jax.ShapeDtypeStruct((n, d), jnp.float32)],
    )(data, merge, w1t, b1, w2t, b2)


def _combine(out, skip, partials, degs, w1t, b1, n, d, n_pad):
    br = 1000

    def body(out_b, skip_b, p0_b, p1_b, d0_b, d1_b, w1t_b, b1_b, o_b):
        agg = p0_b[0] + p1_b[0]
        deg = d0_b[0, :, :1] + d1_b[0, :, :1]
        msg = jnp.dot(agg, w1t_b[...],
                      preferred_element_type=jnp.float32,
                      precision=lax.Precision.HIGHEST) + deg * b1_b[...]
        lap = (deg * out_b[...] - msg) / jnp.maximum(deg, 1.0)
        o_b[...] = jnp.maximum(lap + skip_b[...], 0.0)

    full = lambda shape: pl.BlockSpec(shape, lambda i: tuple(0 for _ in shape))
    return pl.pallas_call(
        body,
        grid=(n // br,),
        in_specs=[
            pl.BlockSpec((br, d), lambda i: (i, 0)),
            pl.BlockSpec((br, d), lambda i: (i, 0)),
            pl.BlockSpec((1, br, d), lambda i: (0, i, 0)),
            pl.BlockSpec((1, br, d), lambda i: (1, i, 0)),
            pl.BlockSpec((1, br, DW), lambda i: (0, i, 0)),
            pl.BlockSpec((1, br, DW), lambda i: (1, i, 0)),
            full((d, d)), full((1, d)),
        ],
        out_specs=pl.BlockSpec((br, d), lambda i: (i, 0)),
        out_shape=jax.ShapeDtypeStruct((n, d), jnp.float32),
    )(out, skip, partials, partials, degs, degs, w1t, b1)


def kernel(data, merge, structure, W_lin, b_lin, W_tr, b_tr):
    n, d = data.shape
    n_pad = ((n + 127) // 128) * 128

    w1t = W_lin.T
    w2t = W_tr.T
    out, skip = _mm(data, merge, w1t, b_lin[None, :], w2t, b_tr[None, :], n, d)
    partials, degs = _sc_segment_sum(data, structure, n_pad, d)
    return _combine(out, skip, partials, degs,
                    w1t, b_lin[None, :], n, d, n_pad)
